# baseline (speedup 1.0000x reference)
import functools

import jax
import jax.numpy as jnp
from jax.experimental import pallas as pl
from jax.experimental.pallas import tpu as pltpu
from jax.experimental.pallas import tpu_sc as plsc

B = 256
C = 19
T = 1024
BB = 16
R = BB * C
NCOL = 4 * C
NE = 2 * C + C
SUBB = 8
RS = SUBB * C
EPAD = 64
APAD = 368


def _adj_sc_body(src_hbm, dst_hbm, out_hbm, src_v, dst_v, adj_v):
    wid = jax.lax.axis_index("s") * 2 + jax.lax.axis_index("c")

    @pl.when(wid == 0)
    def _():
        pltpu.sync_copy(src_hbm, src_v)
        pltpu.sync_copy(dst_hbm, dst_v)
        for i in range(APAD // 16):
            adj_v[pl.ds(i * 16, 16)] = jnp.zeros((16,), jnp.float32)
        ones = jnp.ones((16,), jnp.float32)
        for i in range(EPAD // 16):
            s = src_v[pl.ds(i * 16, 16)]
            d = dst_v[pl.ds(i * 16, 16)]
            lanei = jax.lax.iota(jnp.int32, 16) + i * 16
            plsc.addupdate_scatter(adj_v, [d * C + s], ones, mask=lanei < NE)
        pltpu.sync_copy(adj_v, out_hbm)


def _adj_sc(src64, dst64):
    mesh = plsc.VectorSubcoreMesh(core_axis_name="c", subcore_axis_name="s")
    return pl.kernel(
        _adj_sc_body,
        mesh=mesh,
        compiler_params=pltpu.CompilerParams(needs_layout_passes=False),
        out_type=jax.ShapeDtypeStruct((APAD,), jnp.float32),
        scratch_types=[
            pltpu.VMEM((EPAD,), jnp.int32),
            pltpu.VMEM((EPAD,), jnp.int32),
            pltpu.VMEM((APAD,), jnp.float32),
        ],
    )(src64, dst64)


def _shift(v, d, axis):
    if d == 0:
        return v
    n = v.shape[axis]
    dd = d % n
    if axis == 0:
        return jnp.concatenate([v[dd:, :], v[:dd, :]], axis=0)
    return jnp.concatenate([v[:, dd:], v[:, :dd]], axis=1)


def _fused_body(x_ref, wg_ref, adj_ref, wf2_ref, bg_ref, bf_ref,
                wt_ref, bt_ref, wr_ref, br_ref, out_ref, k_ref, ak_ref):
    f32 = jnp.float32

    @pl.when(pl.program_id(0) == 0)
    def _build_constants():
        kri = jax.lax.broadcasted_iota(jnp.int32, (T, 2 * T), 0)
        kci = jax.lax.broadcasted_iota(jnp.int32, (T, 2 * T), 1)
        kd = kri - (kci % T)
        kacc = jnp.zeros((T, 2 * T), f32)
        for d in range(-4, 5):
            w = jnp.where(kci < T, wt_ref[0, d + 4], wt_ref[1, d + 4])
            kacc = kacc + jnp.where(kd == d, w, 0.0)
        k_ref[...] = kacc
        adj = adj_ref[...]
        deg = jnp.sum(adj, axis=1, keepdims=True)
        r = jax.lax.rsqrt(deg)
        ri = jax.lax.broadcasted_iota(jnp.int32, (C, C), 0)
        ci = jax.lax.broadcasted_iota(jnp.int32, (C, C), 1)
        dmat = jnp.where(ri == ci, jnp.broadcast_to(r, (C, C)), 0.0)
        a = jnp.dot(dmat, jnp.dot(adj, dmat, preferred_element_type=f32),
                    preferred_element_type=f32)
        arows = jnp.concatenate([a] * SUBB, axis=0)
        afull = jnp.concatenate([arows] * SUBB, axis=1)
        rri = jax.lax.broadcasted_iota(jnp.int32, (RS, RS), 0) // C
        cci = jax.lax.broadcasted_iota(jnp.int32, (RS, RS), 1) // C
        ak_ref[...] = jnp.where(rri == cci, afull, 0.0)

    xb = x_ref[...].reshape(R, T)
    lane = jax.lax.broadcasted_iota(jnp.int32, (R, T), 1)

    lane2 = jax.lax.broadcasted_iota(jnp.int32, (1, 2 * T), 1)
    btsel = jnp.where(lane2 < T, bt_ref[0], bt_ref[1])
    astart = (0, 128, 384, 512)
    hblocks = []
    for f in range(2):
        for tb in range(4):
            a = astart[tb]
            cb = f * T + tb * 256
            hblocks.append(jnp.dot(xb[:, a:a + 512],
                                   k_ref[a:a + 512, cb:cb + 256],
                                   preferred_element_type=f32))
    h = jnp.maximum(jnp.concatenate(hblocks, axis=1) + btsel, 0.0)

    y = jnp.dot(h, wg_ref[...], preferred_element_type=f32) + bg_ref[...]

    ak = ak_ref[...]
    xs = jnp.maximum(jnp.concatenate(
        [jnp.dot(ak, y[i * RS:(i + 1) * RS, :], preferred_element_type=f32)
         for i in range(BB // SUBB)], axis=0), 0.0)

    cpos = jax.lax.broadcasted_iota(jnp.int32, (R, T), 0) % C
    racc = jnp.zeros((R, T), f32)
    for dc in (-1, 0, 1):
        s = _shift(xs, dc, 0)
        if dc < 0:
            s = jnp.where(cpos >= -dc, s, 0.0)
        elif dc > 0:
            s = jnp.where(cpos < C - dc, s, 0.0)
        for dt in (-1, 0, 1):
            s2 = _shift(s, dt, 1)
            if dt < 0:
                s2 = jnp.where(lane >= -dt, s2, 0.0)
            elif dt > 0:
                s2 = jnp.where(lane < T - dt, s2, 0.0)
            racc = racc + wr_ref[dc + 1, dt + 1] * s2
    xres = jnp.maximum(xs + racc + br_ref[0], 0.0)

    p = jnp.dot(xres, wf2_ref[...], preferred_element_type=f32)
    rp = jax.lax.broadcasted_iota(jnp.int32, (R, NCOL), 0) % C
    cp = jax.lax.broadcasted_iota(jnp.int32, (R, NCOL), 1) // 4
    dsel = jnp.where(rp == cp, p, 0.0)
    s4r = jax.lax.broadcasted_iota(jnp.int32, (NCOL, 4), 0) % 4
    s4c = jax.lax.broadcasted_iota(jnp.int32, (NCOL, 4), 1)
    sel4 = jnp.where(s4r == s4c, 1.0, 0.0).astype(f32)
    q = jnp.dot(dsel, sel4, preferred_element_type=f32)
    gr = jax.lax.broadcasted_iota(jnp.int32, (BB, R), 0)
    gc = jax.lax.broadcasted_iota(jnp.int32, (BB, R), 1) // C
    gsum = jnp.where(gr == gc, 1.0, 0.0).astype(f32)
    out_ref[...] = jnp.dot(gsum, q, preferred_element_type=f32) + bf_ref[...]


@functools.partial(jax.jit, static_argnames=())
def _fused(xr, wg, adj, wf2, bg2, bf2, wt2, bt, wr2, br):
    grid = (B // BB,)
    return pl.pallas_call(
        _fused_body,
        grid=grid,
        in_specs=[
            pl.BlockSpec((BB, C, T), lambda i: (i, 0, 0)),
            pl.BlockSpec((2 * T, T), lambda i: (0, 0)),
            pl.BlockSpec((C, C), lambda i: (0, 0)),
            pl.BlockSpec((T, NCOL), lambda i: (0, 0)),
            pl.BlockSpec((1, T), lambda i: (0, 0)),
            pl.BlockSpec((1, 4), lambda i: (0, 0)),
            pl.BlockSpec(memory_space=pltpu.SMEM),
            pl.BlockSpec(memory_space=pltpu.SMEM),
            pl.BlockSpec(memory_space=pltpu.SMEM),
            pl.BlockSpec(memory_space=pltpu.SMEM),
        ],
        out_specs=pl.BlockSpec((BB, 4), lambda i: (i, 0)),
        out_shape=jax.ShapeDtypeStruct((B, 4), jnp.float32),
        scratch_shapes=[
            pltpu.VMEM((T, 2 * T), jnp.float32),
            pltpu.VMEM((RS, RS), jnp.float32),
        ],
        compiler_params=pltpu.CompilerParams(
            dimension_semantics=("arbitrary",),
        ),
    )(xr, wg, adj, wf2, bg2, bf2, wt2, bt, wr2, br)


def kernel(x, Wt, bt, Wg, bg, Wr, br, Wf, bf, edge_index):
    xr = x.reshape(B, C, T)
    wt2 = Wt.reshape(2, 9)
    wr2 = Wr.reshape(3, 3)
    wf2 = jnp.transpose(Wf.reshape(C, T, 4), (1, 0, 2)).reshape(T, NCOL)
    bg2 = bg.reshape(1, T)
    bf2 = bf.reshape(1, 4)
    self_loop = jnp.arange(C, dtype=jnp.int32)
    pad = jnp.zeros((EPAD - NE,), jnp.int32)
    src64 = jnp.concatenate([edge_index[0].astype(jnp.int32), self_loop, pad])
    dst64 = jnp.concatenate([edge_index[1].astype(jnp.int32), self_loop, pad])
    adj = _adj_sc(src64, dst64)[: C * C].reshape(C, C)
    return _fused(xr, Wg, adj, wf2, bg2, bf2, wt2, bt, wr2, br)


# speedup vs baseline: 1.0395x; 1.0395x over previous
import functools

import jax
import jax.numpy as jnp
from jax.experimental import pallas as pl
from jax.experimental.pallas import tpu as pltpu
from jax.experimental.pallas import tpu_sc as plsc

B = 256
C = 19
T = 1024
BB = 32
R = BB * C
NCOL = 4 * C
NE = 2 * C + C
SUBB = 8
RS = SUBB * C
EPAD = 64
APAD = 368


def _adj_sc_body(src_hbm, dst_hbm, out_hbm, src_v, dst_v, adj_v):
    wid = jax.lax.axis_index("s") * 2 + jax.lax.axis_index("c")

    @pl.when(wid == 0)
    def _():
        pltpu.sync_copy(src_hbm, src_v)
        pltpu.sync_copy(dst_hbm, dst_v)
        for i in range(APAD // 16):
            adj_v[pl.ds(i * 16, 16)] = jnp.zeros((16,), jnp.float32)
        ones = jnp.ones((16,), jnp.float32)
        for i in range(EPAD // 16):
            s = src_v[pl.ds(i * 16, 16)]
            d = dst_v[pl.ds(i * 16, 16)]
            lanei = jax.lax.iota(jnp.int32, 16) + i * 16
            plsc.addupdate_scatter(adj_v, [d * C + s], ones, mask=lanei < NE)
        pltpu.sync_copy(adj_v, out_hbm)


def _adj_sc(src64, dst64):
    mesh = plsc.VectorSubcoreMesh(core_axis_name="c", subcore_axis_name="s")
    return pl.kernel(
        _adj_sc_body,
        mesh=mesh,
        compiler_params=pltpu.CompilerParams(needs_layout_passes=False),
        out_type=jax.ShapeDtypeStruct((APAD,), jnp.float32),
        scratch_types=[
            pltpu.VMEM((EPAD,), jnp.int32),
            pltpu.VMEM((EPAD,), jnp.int32),
            pltpu.VMEM((APAD,), jnp.float32),
        ],
    )(src64, dst64)


def _shift(v, d, axis):
    if d == 0:
        return v
    n = v.shape[axis]
    dd = d % n
    if axis == 0:
        return jnp.concatenate([v[dd:, :], v[:dd, :]], axis=0)
    return jnp.concatenate([v[:, dd:], v[:, :dd]], axis=1)


def _fused_body(x_ref, wg_ref, adj_ref, wf2_ref, bg_ref, bf_ref,
                wt_ref, bt_ref, wr_ref, br_ref, out_ref, k_ref, ak_ref):
    f32 = jnp.float32

    @pl.when(pl.program_id(0) == 0)
    def _build_constants():
        kri = jax.lax.broadcasted_iota(jnp.int32, (T, 2 * T), 0)
        kci = jax.lax.broadcasted_iota(jnp.int32, (T, 2 * T), 1)
        kd = kri - (kci % T)
        kacc = jnp.zeros((T, 2 * T), f32)
        for d in range(-4, 5):
            w = jnp.where(kci < T, wt_ref[0, d + 4], wt_ref[1, d + 4])
            kacc = kacc + jnp.where(kd == d, w, 0.0)
        k_ref[...] = kacc
        adj = adj_ref[...]
        deg = jnp.sum(adj, axis=1, keepdims=True)
        r = jax.lax.rsqrt(deg)
        ri = jax.lax.broadcasted_iota(jnp.int32, (C, C), 0)
        ci = jax.lax.broadcasted_iota(jnp.int32, (C, C), 1)
        dmat = jnp.where(ri == ci, jnp.broadcast_to(r, (C, C)), 0.0)
        a = jnp.dot(dmat, jnp.dot(adj, dmat, preferred_element_type=f32),
                    preferred_element_type=f32)
        arows = jnp.concatenate([a] * SUBB, axis=0)
        afull = jnp.concatenate([arows] * SUBB, axis=1)
        rri = jax.lax.broadcasted_iota(jnp.int32, (RS, RS), 0) // C
        cci = jax.lax.broadcasted_iota(jnp.int32, (RS, RS), 1) // C
        ak_ref[...] = jnp.where(rri == cci, afull, 0.0)

    xb = x_ref[...].reshape(R, T)
    lane = jax.lax.broadcasted_iota(jnp.int32, (R, T), 1)

    lane2 = jax.lax.broadcasted_iota(jnp.int32, (1, 2 * T), 1)
    btsel = jnp.where(lane2 < T, bt_ref[0], bt_ref[1])
    astart = (0, 128, 384, 512)
    hblocks = []
    for f in range(2):
        for tb in range(4):
            a = astart[tb]
            cb = f * T + tb * 256
            hblocks.append(jnp.dot(xb[:, a:a + 512],
                                   k_ref[a:a + 512, cb:cb + 256],
                                   preferred_element_type=f32))
    h = jnp.maximum(jnp.concatenate(hblocks, axis=1) + btsel, 0.0)

    y = jnp.dot(h, wg_ref[...], preferred_element_type=f32) + bg_ref[...]

    ak = ak_ref[...]
    y4 = jnp.concatenate([y[i * RS:(i + 1) * RS, :]
                          for i in range(BB // SUBB)], axis=1)
    z4 = jnp.dot(ak, y4, preferred_element_type=f32)
    xs = jnp.maximum(jnp.concatenate(
        [z4[:, i * T:(i + 1) * T] for i in range(BB // SUBB)], axis=0), 0.0)

    cpos = jax.lax.broadcasted_iota(jnp.int32, (R, T), 0) % C
    racc = jnp.zeros((R, T), f32)
    for dc in (-1, 0, 1):
        s = _shift(xs, dc, 0)
        if dc < 0:
            s = jnp.where(cpos >= -dc, s, 0.0)
        elif dc > 0:
            s = jnp.where(cpos < C - dc, s, 0.0)
        for dt in (-1, 0, 1):
            s2 = _shift(s, dt, 1)
            if dt < 0:
                s2 = jnp.where(lane >= -dt, s2, 0.0)
            elif dt > 0:
                s2 = jnp.where(lane < T - dt, s2, 0.0)
            racc = racc + wr_ref[dc + 1, dt + 1] * s2
    xres = jnp.maximum(xs + racc + br_ref[0], 0.0)

    p = jnp.dot(xres, wf2_ref[...], preferred_element_type=f32)
    rp = jax.lax.broadcasted_iota(jnp.int32, (R, NCOL), 0) % C
    cp = jax.lax.broadcasted_iota(jnp.int32, (R, NCOL), 1) // 4
    dsel = jnp.where(rp == cp, p, 0.0)
    s4r = jax.lax.broadcasted_iota(jnp.int32, (NCOL, 4), 0) % 4
    s4c = jax.lax.broadcasted_iota(jnp.int32, (NCOL, 4), 1)
    sel4 = jnp.where(s4r == s4c, 1.0, 0.0).astype(f32)
    q = jnp.dot(dsel, sel4, preferred_element_type=f32)
    gr = jax.lax.broadcasted_iota(jnp.int32, (BB, R), 0)
    gc = jax.lax.broadcasted_iota(jnp.int32, (BB, R), 1) // C
    gsum = jnp.where(gr == gc, 1.0, 0.0).astype(f32)
    out_ref[...] = jnp.dot(gsum, q, preferred_element_type=f32) + bf_ref[...]


@functools.partial(jax.jit, static_argnames=())
def _fused(xr, wg, adj, wf2, bg2, bf2, wt2, bt, wr2, br):
    grid = (B // BB,)
    return pl.pallas_call(
        _fused_body,
        grid=grid,
        in_specs=[
            pl.BlockSpec((BB, C, T), lambda i: (i, 0, 0)),
            pl.BlockSpec((2 * T, T), lambda i: (0, 0)),
            pl.BlockSpec((C, C), lambda i: (0, 0)),
            pl.BlockSpec((T, NCOL), lambda i: (0, 0)),
            pl.BlockSpec((1, T), lambda i: (0, 0)),
            pl.BlockSpec((1, 4), lambda i: (0, 0)),
            pl.BlockSpec(memory_space=pltpu.SMEM),
            pl.BlockSpec(memory_space=pltpu.SMEM),
            pl.BlockSpec(memory_space=pltpu.SMEM),
            pl.BlockSpec(memory_space=pltpu.SMEM),
        ],
        out_specs=pl.BlockSpec((BB, 4), lambda i: (i, 0)),
        out_shape=jax.ShapeDtypeStruct((B, 4), jnp.float32),
        scratch_shapes=[
            pltpu.VMEM((T, 2 * T), jnp.float32),
            pltpu.VMEM((RS, RS), jnp.float32),
        ],
        compiler_params=pltpu.CompilerParams(
            dimension_semantics=("arbitrary",),
        ),
    )(xr, wg, adj, wf2, bg2, bf2, wt2, bt, wr2, br)


def kernel(x, Wt, bt, Wg, bg, Wr, br, Wf, bf, edge_index):
    xr = x.reshape(B, C, T)
    wt2 = Wt.reshape(2, 9)
    wr2 = Wr.reshape(3, 3)
    wf2 = jnp.transpose(Wf.reshape(C, T, 4), (1, 0, 2)).reshape(T, NCOL)
    bg2 = bg.reshape(1, T)
    bf2 = bf.reshape(1, 4)
    self_loop = jnp.arange(C, dtype=jnp.int32)
    pad = jnp.zeros((EPAD - NE,), jnp.int32)
    src64 = jnp.concatenate([edge_index[0].astype(jnp.int32), self_loop, pad])
    dst64 = jnp.concatenate([edge_index[1].astype(jnp.int32), self_loop, pad])
    adj = _adj_sc(src64, dst64)[: C * C].reshape(C, C)
    return _fused(xr, Wg, adj, wf2, bg2, bf2, wt2, bt, wr2, br)


# speedup vs baseline: 1.2196x; 1.1732x over previous
import functools

import jax
import jax.numpy as jnp
from jax.experimental import pallas as pl
from jax.experimental.pallas import tpu as pltpu
from jax.experimental.pallas import tpu_sc as plsc

B = 256
C = 19
T = 1024
BB = 32
R = BB * C
NCOL = 4 * C
NE = 2 * C + C
SUBB = 8
RS = SUBB * C
EPAD = 64
APAD = 368


def _adj_sc_body(src_hbm, dst_hbm, out_hbm, src_v, dst_v, adj_v):
    wid = jax.lax.axis_index("s") * 2 + jax.lax.axis_index("c")

    @pl.when(wid == 0)
    def _():
        pltpu.sync_copy(src_hbm, src_v)
        pltpu.sync_copy(dst_hbm, dst_v)
        for i in range(APAD // 16):
            adj_v[pl.ds(i * 16, 16)] = jnp.zeros((16,), jnp.float32)
        ones = jnp.ones((16,), jnp.float32)
        for i in range(EPAD // 16):
            s = src_v[pl.ds(i * 16, 16)]
            d = dst_v[pl.ds(i * 16, 16)]
            lanei = jax.lax.iota(jnp.int32, 16) + i * 16
            plsc.addupdate_scatter(adj_v, [d * C + s], ones, mask=lanei < NE)
        pltpu.sync_copy(adj_v, out_hbm)


def _adj_sc(src64, dst64):
    mesh = plsc.VectorSubcoreMesh(core_axis_name="c", subcore_axis_name="s")
    return pl.kernel(
        _adj_sc_body,
        mesh=mesh,
        compiler_params=pltpu.CompilerParams(needs_layout_passes=False),
        out_type=jax.ShapeDtypeStruct((APAD,), jnp.float32),
        scratch_types=[
            pltpu.VMEM((EPAD,), jnp.int32),
            pltpu.VMEM((EPAD,), jnp.int32),
            pltpu.VMEM((APAD,), jnp.float32),
        ],
    )(src64, dst64)


def _shift(v, d, axis):
    if d == 0:
        return v
    n = v.shape[axis]
    dd = d % n
    if axis == 0:
        return jnp.concatenate([v[dd:, :], v[:dd, :]], axis=0)
    return jnp.concatenate([v[:, dd:], v[:, :dd]], axis=1)


def _fused_body(x_ref, wg_ref, adj_ref, wf2_ref, bg_ref, bf_ref,
                wt_ref, bt_ref, wr_ref, br_ref, out_ref, k_ref, ak_ref,
                mr_ref):
    f32 = jnp.float32

    @pl.when(pl.program_id(0) == 0)
    def _build_constants():
        kri = jax.lax.broadcasted_iota(jnp.int32, (T, 2 * T), 0)
        kci = jax.lax.broadcasted_iota(jnp.int32, (T, 2 * T), 1)
        kd = kri - (kci % T)
        kacc = jnp.zeros((T, 2 * T), f32)
        for d in range(-4, 5):
            w = jnp.where(kci < T, wt_ref[0, d + 4], wt_ref[1, d + 4])
            kacc = kacc + jnp.where(kd == d, w, 0.0)
        k_ref[...] = kacc
        adj = adj_ref[...]
        deg = jnp.sum(adj, axis=1, keepdims=True)
        r = jax.lax.rsqrt(deg)
        ri = jax.lax.broadcasted_iota(jnp.int32, (C, C), 0)
        ci = jax.lax.broadcasted_iota(jnp.int32, (C, C), 1)
        dmat = jnp.where(ri == ci, jnp.broadcast_to(r, (C, C)), 0.0)
        a = jnp.dot(dmat, jnp.dot(adj, dmat, preferred_element_type=f32),
                    preferred_element_type=f32)
        arows = jnp.concatenate([a] * SUBB, axis=0)
        afull = jnp.concatenate([arows] * SUBB, axis=1)
        rri = jax.lax.broadcasted_iota(jnp.int32, (RS, RS), 0) // C
        cci = jax.lax.broadcasted_iota(jnp.int32, (RS, RS), 1) // C
        ak_ref[...] = jnp.where(rri == cci, afull, 0.0)
        rr = jax.lax.broadcasted_iota(jnp.int32, (RS, RS), 0)
        cc = jax.lax.broadcasted_iota(jnp.int32, (RS, RS), 1)
        dcm = cc - rr
        sameblk = (rri == cci) & (dcm >= -1) & (dcm <= 1)
        for j, dt in enumerate((-1, 0, 1)):
            wsel = jnp.where(
                dcm == -1, wr_ref[0, dt + 1],
                jnp.where(dcm == 0, wr_ref[1, dt + 1], wr_ref[2, dt + 1]))
            mr_ref[:, j * RS:(j + 1) * RS] = jnp.where(sameblk, wsel, 0.0)

    xb = x_ref[...].reshape(R, T)
    lane = jax.lax.broadcasted_iota(jnp.int32, (R, T), 1)

    lane2 = jax.lax.broadcasted_iota(jnp.int32, (1, 2 * T), 1)
    btsel = jnp.where(lane2 < T, bt_ref[0], bt_ref[1])
    astart = (0, 128, 384, 512)
    hblocks = []
    for f in range(2):
        for tb in range(4):
            a = astart[tb]
            cb = f * T + tb * 256
            hblocks.append(jnp.dot(xb[:, a:a + 512],
                                   k_ref[a:a + 512, cb:cb + 256],
                                   preferred_element_type=f32))
    h = jnp.maximum(jnp.concatenate(hblocks, axis=1) + btsel, 0.0)

    y = jnp.dot(h, wg_ref[...], preferred_element_type=f32) + bg_ref[...]

    ak = ak_ref[...]
    xs = jnp.maximum(jnp.concatenate(
        [jnp.dot(ak, y[i * RS:(i + 1) * RS, :], preferred_element_type=f32)
         for i in range(BB // SUBB)], axis=0), 0.0)

    sdt = []
    for dt in (-1, 0, 1):
        s2 = _shift(xs, dt, 1)
        if dt < 0:
            s2 = jnp.where(lane >= -dt, s2, 0.0)
        elif dt > 0:
            s2 = jnp.where(lane < T - dt, s2, 0.0)
        sdt.append(s2)
    slices = []
    for i in range(BB // SUBB):
        acc = None
        for j in range(3):
            d = jnp.dot(mr_ref[:, j * RS:(j + 1) * RS],
                        sdt[j][i * RS:(i + 1) * RS, :],
                        preferred_element_type=f32)
            acc = d if acc is None else acc + d
        slices.append(acc)
    racc = jnp.concatenate(slices, axis=0)
    xres = jnp.maximum(xs + racc + br_ref[0], 0.0)

    p = jnp.dot(xres, wf2_ref[...], preferred_element_type=f32)
    rp = jax.lax.broadcasted_iota(jnp.int32, (R, NCOL), 0) % C
    cp = jax.lax.broadcasted_iota(jnp.int32, (R, NCOL), 1) // 4
    dsel = jnp.where(rp == cp, p, 0.0)
    s4r = jax.lax.broadcasted_iota(jnp.int32, (NCOL, 4), 0) % 4
    s4c = jax.lax.broadcasted_iota(jnp.int32, (NCOL, 4), 1)
    sel4 = jnp.where(s4r == s4c, 1.0, 0.0).astype(f32)
    q = jnp.dot(dsel, sel4, preferred_element_type=f32)
    gr = jax.lax.broadcasted_iota(jnp.int32, (BB, R), 0)
    gc = jax.lax.broadcasted_iota(jnp.int32, (BB, R), 1) // C
    gsum = jnp.where(gr == gc, 1.0, 0.0).astype(f32)
    out_ref[...] = jnp.dot(gsum, q, preferred_element_type=f32) + bf_ref[...]


@functools.partial(jax.jit, static_argnames=())
def _fused(xr, wg, adj, wf2, bg2, bf2, wt2, bt, wr2, br):
    grid = (B // BB,)
    return pl.pallas_call(
        _fused_body,
        grid=grid,
        in_specs=[
            pl.BlockSpec((BB, C, T), lambda i: (i, 0, 0)),
            pl.BlockSpec((2 * T, T), lambda i: (0, 0)),
            pl.BlockSpec((C, C), lambda i: (0, 0)),
            pl.BlockSpec((T, NCOL), lambda i: (0, 0)),
            pl.BlockSpec((1, T), lambda i: (0, 0)),
            pl.BlockSpec((1, 4), lambda i: (0, 0)),
            pl.BlockSpec(memory_space=pltpu.SMEM),
            pl.BlockSpec(memory_space=pltpu.SMEM),
            pl.BlockSpec(memory_space=pltpu.SMEM),
            pl.BlockSpec(memory_space=pltpu.SMEM),
        ],
        out_specs=pl.BlockSpec((BB, 4), lambda i: (i, 0)),
        out_shape=jax.ShapeDtypeStruct((B, 4), jnp.float32),
        scratch_shapes=[
            pltpu.VMEM((T, 2 * T), jnp.float32),
            pltpu.VMEM((RS, RS), jnp.float32),
            pltpu.VMEM((RS, 3 * RS), jnp.float32),
        ],
        compiler_params=pltpu.CompilerParams(
            dimension_semantics=("arbitrary",),
        ),
    )(xr, wg, adj, wf2, bg2, bf2, wt2, bt, wr2, br)


def kernel(x, Wt, bt, Wg, bg, Wr, br, Wf, bf, edge_index):
    xr = x.reshape(B, C, T)
    wt2 = Wt.reshape(2, 9)
    wr2 = Wr.reshape(3, 3)
    wf2 = jnp.transpose(Wf.reshape(C, T, 4), (1, 0, 2)).reshape(T, NCOL)
    bg2 = bg.reshape(1, T)
    bf2 = bf.reshape(1, 4)
    self_loop = jnp.arange(C, dtype=jnp.int32)
    pad = jnp.zeros((EPAD - NE,), jnp.int32)
    src64 = jnp.concatenate([edge_index[0].astype(jnp.int32), self_loop, pad])
    dst64 = jnp.concatenate([edge_index[1].astype(jnp.int32), self_loop, pad])
    adj = _adj_sc(src64, dst64)[: C * C].reshape(C, C)
    return _fused(xr, Wg, adj, wf2, bg2, bf2, wt2, bt, wr2, br)


# speedup vs baseline: 1.2382x; 1.0152x over previous
import functools

import jax
import jax.numpy as jnp
from jax.experimental import pallas as pl
from jax.experimental.pallas import tpu as pltpu
from jax.experimental.pallas import tpu_sc as plsc

B = 256
C = 19
T = 1024
BB = 64
R = BB * C
NCOL = 4 * C
NE = 2 * C + C
SUBB = 8
RS = SUBB * C
EPAD = 64
APAD = 368


def _adj_sc_body(src_hbm, dst_hbm, out_hbm, src_v, dst_v, adj_v):
    wid = jax.lax.axis_index("s") * 2 + jax.lax.axis_index("c")

    @pl.when(wid == 0)
    def _():
        pltpu.sync_copy(src_hbm, src_v)
        pltpu.sync_copy(dst_hbm, dst_v)
        for i in range(APAD // 16):
            adj_v[pl.ds(i * 16, 16)] = jnp.zeros((16,), jnp.float32)
        ones = jnp.ones((16,), jnp.float32)
        for i in range(EPAD // 16):
            s = src_v[pl.ds(i * 16, 16)]
            d = dst_v[pl.ds(i * 16, 16)]
            lanei = jax.lax.iota(jnp.int32, 16) + i * 16
            plsc.addupdate_scatter(adj_v, [d * C + s], ones, mask=lanei < NE)
        pltpu.sync_copy(adj_v, out_hbm)


def _adj_sc(src64, dst64):
    mesh = plsc.VectorSubcoreMesh(core_axis_name="c", subcore_axis_name="s")
    return pl.kernel(
        _adj_sc_body,
        mesh=mesh,
        compiler_params=pltpu.CompilerParams(needs_layout_passes=False),
        out_type=jax.ShapeDtypeStruct((APAD,), jnp.float32),
        scratch_types=[
            pltpu.VMEM((EPAD,), jnp.int32),
            pltpu.VMEM((EPAD,), jnp.int32),
            pltpu.VMEM((APAD,), jnp.float32),
        ],
    )(src64, dst64)


def _shift(v, d, axis):
    if d == 0:
        return v
    n = v.shape[axis]
    dd = d % n
    if axis == 0:
        return jnp.concatenate([v[dd:, :], v[:dd, :]], axis=0)
    return jnp.concatenate([v[:, dd:], v[:, :dd]], axis=1)


def _fused_body(x_ref, wg_ref, adj_ref, wf2_ref, bg_ref, bf_ref,
                wt_ref, bt_ref, wr_ref, br_ref, out_ref, k_ref, ak_ref,
                mr_ref):
    f32 = jnp.float32

    @pl.when(pl.program_id(0) == 0)
    def _build_constants():
        kri = jax.lax.broadcasted_iota(jnp.int32, (T, 2 * T), 0)
        kci = jax.lax.broadcasted_iota(jnp.int32, (T, 2 * T), 1)
        kd = kri - (kci % T)
        kacc = jnp.zeros((T, 2 * T), f32)
        for d in range(-4, 5):
            w = jnp.where(kci < T, wt_ref[0, d + 4], wt_ref[1, d + 4])
            kacc = kacc + jnp.where(kd == d, w, 0.0)
        k_ref[...] = kacc
        adj = adj_ref[...]
        deg = jnp.sum(adj, axis=1, keepdims=True)
        r = jax.lax.rsqrt(deg)
        ri = jax.lax.broadcasted_iota(jnp.int32, (C, C), 0)
        ci = jax.lax.broadcasted_iota(jnp.int32, (C, C), 1)
        dmat = jnp.where(ri == ci, jnp.broadcast_to(r, (C, C)), 0.0)
        a = jnp.dot(dmat, jnp.dot(adj, dmat, preferred_element_type=f32),
                    preferred_element_type=f32)
        arows = jnp.concatenate([a] * SUBB, axis=0)
        afull = jnp.concatenate([arows] * SUBB, axis=1)
        rri = jax.lax.broadcasted_iota(jnp.int32, (RS, RS), 0) // C
        cci = jax.lax.broadcasted_iota(jnp.int32, (RS, RS), 1) // C
        ak_ref[...] = jnp.where(rri == cci, afull, 0.0)
        rr = jax.lax.broadcasted_iota(jnp.int32, (RS, RS), 0)
        cc = jax.lax.broadcasted_iota(jnp.int32, (RS, RS), 1)
        dcm = cc - rr
        sameblk = (rri == cci) & (dcm >= -1) & (dcm <= 1)
        for j, dt in enumerate((-1, 0, 1)):
            wsel = jnp.where(
                dcm == -1, wr_ref[0, dt + 1],
                jnp.where(dcm == 0, wr_ref[1, dt + 1], wr_ref[2, dt + 1]))
            mr_ref[:, j * RS:(j + 1) * RS] = jnp.where(sameblk, wsel, 0.0)

    xb = x_ref[...].reshape(R, T)
    lane = jax.lax.broadcasted_iota(jnp.int32, (R, T), 1)

    lane2 = jax.lax.broadcasted_iota(jnp.int32, (1, 2 * T), 1)
    btsel = jnp.where(lane2 < T, bt_ref[0], bt_ref[1])
    astart = (0, 128, 384, 512)
    hblocks = []
    for f in range(2):
        for tb in range(4):
            a = astart[tb]
            cb = f * T + tb * 256
            hblocks.append(jnp.dot(xb[:, a:a + 512],
                                   k_ref[a:a + 512, cb:cb + 256],
                                   preferred_element_type=f32))
    h = jnp.maximum(jnp.concatenate(hblocks, axis=1) + btsel, 0.0)

    y = jnp.dot(h, wg_ref[...], preferred_element_type=f32) + bg_ref[...]

    ak = ak_ref[...]
    xs = jnp.maximum(jnp.concatenate(
        [jnp.dot(ak, y[i * RS:(i + 1) * RS, :], preferred_element_type=f32)
         for i in range(BB // SUBB)], axis=0), 0.0)

    sdt = []
    for dt in (-1, 0, 1):
        s2 = _shift(xs, dt, 1)
        if dt < 0:
            s2 = jnp.where(lane >= -dt, s2, 0.0)
        elif dt > 0:
            s2 = jnp.where(lane < T - dt, s2, 0.0)
        sdt.append(s2)
    slices = []
    for i in range(BB // SUBB):
        acc = None
        for j in range(3):
            d = jnp.dot(mr_ref[:, j * RS:(j + 1) * RS],
                        sdt[j][i * RS:(i + 1) * RS, :],
                        preferred_element_type=f32)
            acc = d if acc is None else acc + d
        slices.append(acc)
    racc = jnp.concatenate(slices, axis=0)
    xres = jnp.maximum(xs + racc + br_ref[0], 0.0)

    p = jnp.dot(xres, wf2_ref[...], preferred_element_type=f32)
    rp = jax.lax.broadcasted_iota(jnp.int32, (R, NCOL), 0) % C
    cp = jax.lax.broadcasted_iota(jnp.int32, (R, NCOL), 1) // 4
    dsel = jnp.where(rp == cp, p, 0.0)
    s4r = jax.lax.broadcasted_iota(jnp.int32, (NCOL, 4), 0) % 4
    s4c = jax.lax.broadcasted_iota(jnp.int32, (NCOL, 4), 1)
    sel4 = jnp.where(s4r == s4c, 1.0, 0.0).astype(f32)
    q = jnp.dot(dsel, sel4, preferred_element_type=f32)
    gr = jax.lax.broadcasted_iota(jnp.int32, (BB, R), 0)
    gc = jax.lax.broadcasted_iota(jnp.int32, (BB, R), 1) // C
    gsum = jnp.where(gr == gc, 1.0, 0.0).astype(f32)
    out_ref[...] = jnp.dot(gsum, q, preferred_element_type=f32) + bf_ref[...]


@functools.partial(jax.jit, static_argnames=())
def _fused(xr, wg, adj, wf2, bg2, bf2, wt2, bt, wr2, br):
    grid = (B // BB,)
    return pl.pallas_call(
        _fused_body,
        grid=grid,
        in_specs=[
            pl.BlockSpec((BB, C, T), lambda i: (i, 0, 0)),
            pl.BlockSpec((2 * T, T), lambda i: (0, 0)),
            pl.BlockSpec((C, C), lambda i: (0, 0)),
            pl.BlockSpec((T, NCOL), lambda i: (0, 0)),
            pl.BlockSpec((1, T), lambda i: (0, 0)),
            pl.BlockSpec((1, 4), lambda i: (0, 0)),
            pl.BlockSpec(memory_space=pltpu.SMEM),
            pl.BlockSpec(memory_space=pltpu.SMEM),
            pl.BlockSpec(memory_space=pltpu.SMEM),
            pl.BlockSpec(memory_space=pltpu.SMEM),
        ],
        out_specs=pl.BlockSpec((BB, 4), lambda i: (i, 0)),
        out_shape=jax.ShapeDtypeStruct((B, 4), jnp.float32),
        scratch_shapes=[
            pltpu.VMEM((T, 2 * T), jnp.float32),
            pltpu.VMEM((RS, RS), jnp.float32),
            pltpu.VMEM((RS, 3 * RS), jnp.float32),
        ],
        compiler_params=pltpu.CompilerParams(
            dimension_semantics=("arbitrary",),
        ),
    )(xr, wg, adj, wf2, bg2, bf2, wt2, bt, wr2, br)


def kernel(x, Wt, bt, Wg, bg, Wr, br, Wf, bf, edge_index):
    xr = x.reshape(B, C, T)
    wt2 = Wt.reshape(2, 9)
    wr2 = Wr.reshape(3, 3)
    wf2 = jnp.transpose(Wf.reshape(C, T, 4), (1, 0, 2)).reshape(T, NCOL)
    bg2 = bg.reshape(1, T)
    bf2 = bf.reshape(1, 4)
    self_loop = jnp.arange(C, dtype=jnp.int32)
    pad = jnp.zeros((EPAD - NE,), jnp.int32)
    src64 = jnp.concatenate([edge_index[0].astype(jnp.int32), self_loop, pad])
    dst64 = jnp.concatenate([edge_index[1].astype(jnp.int32), self_loop, pad])
    adj = _adj_sc(src64, dst64)[: C * C].reshape(C, C)
    return _fused(xr, Wg, adj, wf2, bg2, bf2, wt2, bt, wr2, br)


# speedup vs baseline: 1.2396x; 1.0011x over previous
import functools

import jax
import jax.numpy as jnp
from jax.experimental import pallas as pl
from jax.experimental.pallas import tpu as pltpu
from jax.experimental.pallas import tpu_sc as plsc

B = 256
C = 19
T = 1024
BB = 64
R = BB * C
NCOL = 4 * C
NE = 2 * C + C
SUBB = 8
RS = SUBB * C
EPAD = 64
APAD = 368


def _adj_sc_body(src_hbm, dst_hbm, out_hbm, src_v, dst_v, adj_v):
    wid = jax.lax.axis_index("s") * 2 + jax.lax.axis_index("c")

    @pl.when(wid == 0)
    def _():
        pltpu.sync_copy(src_hbm, src_v)
        pltpu.sync_copy(dst_hbm, dst_v)
        for i in range(APAD // 16):
            adj_v[pl.ds(i * 16, 16)] = jnp.zeros((16,), jnp.float32)
        ones = jnp.ones((16,), jnp.float32)
        for i in range(EPAD // 16):
            s = src_v[pl.ds(i * 16, 16)]
            d = dst_v[pl.ds(i * 16, 16)]
            lanei = jax.lax.iota(jnp.int32, 16) + i * 16
            plsc.addupdate_scatter(adj_v, [d * C + s], ones, mask=lanei < NE)
        pltpu.sync_copy(adj_v, out_hbm)


def _adj_sc(src64, dst64):
    mesh = plsc.VectorSubcoreMesh(core_axis_name="c", subcore_axis_name="s")
    return pl.kernel(
        _adj_sc_body,
        mesh=mesh,
        compiler_params=pltpu.CompilerParams(needs_layout_passes=False),
        out_type=jax.ShapeDtypeStruct((APAD,), jnp.float32),
        scratch_types=[
            pltpu.VMEM((EPAD,), jnp.int32),
            pltpu.VMEM((EPAD,), jnp.int32),
            pltpu.VMEM((APAD,), jnp.float32),
        ],
    )(src64, dst64)


def _shift(v, d, axis):
    if d == 0:
        return v
    n = v.shape[axis]
    dd = d % n
    if axis == 0:
        return jnp.concatenate([v[dd:, :], v[:dd, :]], axis=0)
    return jnp.concatenate([v[:, dd:], v[:, :dd]], axis=1)


def _fused_body(x_ref, wg_ref, adj_ref, wf2_ref, bg_ref, bf_ref,
                wt_ref, bt_ref, wr_ref, br_ref, out_ref, k_ref, ak_ref,
                mr_ref):
    f32 = jnp.float32

    @pl.when(pl.program_id(0) == 0)
    def _build_constants():
        kri = jax.lax.broadcasted_iota(jnp.int32, (T, 2 * T), 0)
        kci = jax.lax.broadcasted_iota(jnp.int32, (T, 2 * T), 1)
        kd = kri - (kci % T)
        kacc = jnp.zeros((T, 2 * T), f32)
        for d in range(-4, 5):
            w = jnp.where(kci < T, wt_ref[0, d + 4], wt_ref[1, d + 4])
            kacc = kacc + jnp.where(kd == d, w, 0.0)
        k_ref[...] = kacc
        adj = adj_ref[...]
        deg = jnp.sum(adj, axis=1, keepdims=True)
        r = jax.lax.rsqrt(deg)
        ri = jax.lax.broadcasted_iota(jnp.int32, (C, C), 0)
        ci = jax.lax.broadcasted_iota(jnp.int32, (C, C), 1)
        dmat = jnp.where(ri == ci, jnp.broadcast_to(r, (C, C)), 0.0)
        a = jnp.dot(dmat, jnp.dot(adj, dmat, preferred_element_type=f32),
                    preferred_element_type=f32)
        arows = jnp.concatenate([a] * SUBB, axis=0)
        afull = jnp.concatenate([arows] * SUBB, axis=1)
        rri = jax.lax.broadcasted_iota(jnp.int32, (RS, RS), 0) // C
        cci = jax.lax.broadcasted_iota(jnp.int32, (RS, RS), 1) // C
        ak_ref[...] = jnp.where(rri == cci, afull, 0.0)
        rr = jax.lax.broadcasted_iota(jnp.int32, (RS, RS), 0)
        cc = jax.lax.broadcasted_iota(jnp.int32, (RS, RS), 1)
        dcm = cc - rr
        sameblk = (rri == cci) & (dcm >= -1) & (dcm <= 1)
        for j, dt in enumerate((-1, 0, 1)):
            wsel = jnp.where(
                dcm == -1, wr_ref[0, dt + 1],
                jnp.where(dcm == 0, wr_ref[1, dt + 1], wr_ref[2, dt + 1]))
            mr_ref[:, j * RS:(j + 1) * RS] = jnp.where(sameblk, wsel, 0.0)

    xb = x_ref[...].reshape(R, T)
    lane = jax.lax.broadcasted_iota(jnp.int32, (R, T), 1)

    lane2 = jax.lax.broadcasted_iota(jnp.int32, (1, 2 * T), 1)
    btsel = jnp.where(lane2 < T, bt_ref[0], bt_ref[1])
    astart = (0, 128, 384, 512)
    hblocks = []
    for f in range(2):
        for tb in range(4):
            a = astart[tb]
            cb = f * T + tb * 256
            hblocks.append(jnp.dot(xb[:, a:a + 512],
                                   k_ref[a:a + 512, cb:cb + 256],
                                   preferred_element_type=f32))
    h = jnp.maximum(jnp.concatenate(hblocks, axis=1) + btsel, 0.0)

    y = jnp.dot(h, wg_ref[...], preferred_element_type=f32) + bg_ref[...]

    ak = ak_ref[...]
    xs = jnp.maximum(jnp.concatenate(
        [jnp.dot(ak, y[i * RS:(i + 1) * RS, :], preferred_element_type=f32)
         for i in range(BB // SUBB)], axis=0), 0.0)

    sdt = []
    for dt in (-1, 0, 1):
        s2 = _shift(xs, dt, 1)
        if dt < 0:
            s2 = jnp.where(lane >= -dt, s2, 0.0)
        elif dt > 0:
            s2 = jnp.where(lane < T - dt, s2, 0.0)
        sdt.append(s2)
    slices = []
    for i in range(BB // SUBB):
        acc = None
        for j in range(3):
            d = jnp.dot(mr_ref[:, j * RS:(j + 1) * RS],
                        sdt[j][i * RS:(i + 1) * RS, :],
                        preferred_element_type=f32)
            acc = d if acc is None else acc + d
        slices.append(acc)
    racc = jnp.concatenate(slices, axis=0)
    xres = jnp.maximum(xs + racc + br_ref[0], 0.0)

    p = jnp.dot(xres, wf2_ref[...], preferred_element_type=f32)
    rp = jax.lax.broadcasted_iota(jnp.int32, (R, NCOL), 0) % C
    cp = jax.lax.broadcasted_iota(jnp.int32, (R, NCOL), 1) // 4
    dsel = jnp.where(rp == cp, p, 0.0)
    s4r = jax.lax.broadcasted_iota(jnp.int32, (NCOL, 4), 0) % 4
    s4c = jax.lax.broadcasted_iota(jnp.int32, (NCOL, 4), 1)
    sel4 = jnp.where(s4r == s4c, 1.0, 0.0).astype(f32)
    q = jnp.dot(dsel, sel4, preferred_element_type=f32)
    gr = jax.lax.broadcasted_iota(jnp.int32, (BB, R), 0)
    gc = jax.lax.broadcasted_iota(jnp.int32, (BB, R), 1) // C
    gsum = jnp.where(gr == gc, 1.0, 0.0).astype(f32)
    out_ref[...] = jnp.dot(gsum, q, preferred_element_type=f32) + bf_ref[...]


@functools.partial(jax.jit, static_argnames=())
def _fused(xr, wg, adj, wf2, bg2, bf2, wt2, bt, wr2, br):
    grid = (B // BB,)
    return pl.pallas_call(
        _fused_body,
        grid=grid,
        in_specs=[
            pl.BlockSpec((BB, C, T), lambda i: (i, 0, 0)),
            pl.BlockSpec((2 * T, T), lambda i: (0, 0)),
            pl.BlockSpec((C, C), lambda i: (0, 0)),
            pl.BlockSpec((T, NCOL), lambda i: (0, 0)),
            pl.BlockSpec((1, T), lambda i: (0, 0)),
            pl.BlockSpec((1, 4), lambda i: (0, 0)),
            pl.BlockSpec(memory_space=pltpu.SMEM),
            pl.BlockSpec(memory_space=pltpu.SMEM),
            pl.BlockSpec(memory_space=pltpu.SMEM),
            pl.BlockSpec(memory_space=pltpu.SMEM),
        ],
        out_specs=pl.BlockSpec((BB, 4), lambda i: (i, 0)),
        out_shape=jax.ShapeDtypeStruct((B, 4), jnp.float32),
        scratch_shapes=[
            pltpu.VMEM((T, 2 * T), jnp.float32),
            pltpu.VMEM((RS, RS), jnp.float32),
            pltpu.VMEM((RS, 3 * RS), jnp.float32),
        ],
        compiler_params=pltpu.CompilerParams(
            dimension_semantics=("parallel",),
        ),
    )(xr, wg, adj, wf2, bg2, bf2, wt2, bt, wr2, br)


def kernel(x, Wt, bt, Wg, bg, Wr, br, Wf, bf, edge_index):
    xr = x.reshape(B, C, T)
    wt2 = Wt.reshape(2, 9)
    wr2 = Wr.reshape(3, 3)
    wf2 = jnp.transpose(Wf.reshape(C, T, 4), (1, 0, 2)).reshape(T, NCOL)
    bg2 = bg.reshape(1, T)
    bf2 = bf.reshape(1, 4)
    self_loop = jnp.arange(C, dtype=jnp.int32)
    pad = jnp.zeros((EPAD - NE,), jnp.int32)
    src64 = jnp.concatenate([edge_index[0].astype(jnp.int32), self_loop, pad])
    dst64 = jnp.concatenate([edge_index[1].astype(jnp.int32), self_loop, pad])
    adj = _adj_sc(src64, dst64)[: C * C].reshape(C, C)
    return _fused(xr, Wg, adj, wf2, bg2, bf2, wt2, bt, wr2, br)
